# baseline (speedup 1.0000x reference)
import functools

import jax
import jax.numpy as jnp
from jax import lax
from jax.experimental import pallas as pl
from jax.experimental.pallas import tpu as pltpu
from jax.experimental.pallas import tpu_sc as plsc

N = 10000
T = 32
DIN = 16
H = 128
E = 320000
S = 64
EI = 512

NBLK = 1000
NGRID = N // NBLK

NC = 2
NS = 16
NW = NC * NS
EPT = E // NW
KE = 80
NEB = EPT // KE


def _gru_body(xt_ref, wih_ref, whh_ref, bih_ref, bhh_ref, wi_ref, ais_ref,
              aid_ref, seq_ref, xw_ref, as_ref, ad_ref, mx_ref):
    wih = wih_ref[...]
    whh = whh_ref[...]
    bih = bih_ref[...]
    bhh = bhh_ref[...]

    def step(t, h):
        xt = xt_ref[t]
        gi = jnp.dot(xt, wih, preferred_element_type=jnp.float32) + bih
        gh = jnp.dot(h, whh, preferred_element_type=jnp.float32) + bhh
        r = jax.nn.sigmoid(gi[:, :H] + gh[:, :H])
        z = jax.nn.sigmoid(gi[:, H:2 * H] + gh[:, H:2 * H])
        n = jnp.tanh(gi[:, 2 * H:] + r * gh[:, 2 * H:])
        return (1.0 - z) * n + z * h

    h = lax.fori_loop(0, T, step, jnp.zeros((NBLK, H), jnp.float32))
    seq_ref[...] = h
    xw = jnp.dot(h, wi_ref[...], preferred_element_type=jnp.float32)
    xw_ref[0] = xw[:, :H // 2]
    xw_ref[1] = xw[:, H // 2:]
    a_s = jnp.dot(xw, ais_ref[...], preferred_element_type=jnp.float32)
    a_d = jnp.dot(xw, aid_ref[...], preferred_element_type=jnp.float32)
    as_ref[...] = a_s
    ad_ref[...] = a_d
    i = pl.program_id(0)

    @pl.when(i == 0)
    def _():
        mx_ref[...] = jnp.full((1, 1), -jnp.inf, jnp.float32)

    mx_ref[...] = jnp.maximum(mx_ref[...], jnp.full((1, 1), jnp.max(a_s)))


def _run_gru(xt, w_ih, w_hh, b_ih, b_hh, wi, ai_src, ai_dst):
    return pl.pallas_call(
        _gru_body,
        grid=(NGRID,),
        in_specs=[
            pl.BlockSpec((T, NBLK, DIN), lambda i: (0, i, 0)),
            pl.BlockSpec((DIN, 3 * H), lambda i: (0, 0)),
            pl.BlockSpec((H, 3 * H), lambda i: (0, 0)),
            pl.BlockSpec((1, 3 * H), lambda i: (0, 0)),
            pl.BlockSpec((1, 3 * H), lambda i: (0, 0)),
            pl.BlockSpec((H, H), lambda i: (0, 0)),
            pl.BlockSpec((H, 1), lambda i: (0, 0)),
            pl.BlockSpec((H, 1), lambda i: (0, 0)),
        ],
        out_specs=[
            pl.BlockSpec((NBLK, H), lambda i: (i, 0)),
            pl.BlockSpec((2, NBLK, H // 2), lambda i: (0, i, 0)),
            pl.BlockSpec((NBLK, 1), lambda i: (i, 0)),
            pl.BlockSpec((NBLK, 1), lambda i: (i, 0)),
            pl.BlockSpec((1, 1), lambda i: (0, 0)),
        ],
        out_shape=[
            jax.ShapeDtypeStruct((N, H), jnp.float32),
            jax.ShapeDtypeStruct((2, N, H // 2), jnp.float32),
            jax.ShapeDtypeStruct((N, 1), jnp.float32),
            jax.ShapeDtypeStruct((N, 1), jnp.float32),
            jax.ShapeDtypeStruct((1, 1), jnp.float32),
        ],
    )(xt, w_ih, w_hh, b_ih, b_hh, wi, ai_src, ai_dst)


HC = H // 2
EPT2 = E // NS
NB = EPT2 // KE


def _edge_sc_body(ei_hbm, as_hbm, ad_hbm, mx_hbm, xw_hbm,
                  acc_hbm, den_hbm,
                  as_v, ad_v, den_v, s0, d0, s1, d1, r0, r1, e0, e1, mx_v,
                  acc_sh, sg0, sg1, ss0, ss1):
    cid = lax.axis_index("c")
    sid = lax.axis_index("s")
    base = sid * EPT2
    coff = cid * N

    pltpu.sync_copy(as_hbm, as_v)
    pltpu.sync_copy(ad_hbm, ad_v)
    pltpu.sync_copy(mx_hbm, mx_v)
    mxv = mx_v[...]

    def zden(j, c):
        den_v[pl.ds(j * 16, 16)] = jnp.zeros((16,), jnp.float32)
        return c
    lax.fori_loop(0, N // 16, zden, 0)

    def zrows(j, c):
        for cc in range(HC // 16):
            r0[j, pl.ds(cc * 16, 16)] = jnp.zeros((16,), jnp.float32)
        return c
    lax.fori_loop(0, KE, zrows, 0)

    @pl.when(sid == 0)
    def _():
        def zacc(b, c):
            pltpu.sync_copy(r0, acc_sh.at[pl.ds(b * KE, KE)])
            return c
        lax.fori_loop(0, N // KE, zacc, 0)

    plsc.subcore_barrier()

    def load_idx(b, s_v, d_v):
        off = base + b * KE
        pltpu.sync_copy(ei_hbm.at[pl.ds(off, KE)], s_v)
        pltpu.sync_copy(ei_hbm.at[pl.ds(E + off, KE)], d_v)
        for g in range(KE // 16):
            s_v[pl.ds(g * 16, 16)] = s_v[pl.ds(g * 16, 16)] + coff

    def compute_ex(s_v, d_v, e_v):
        for g in range(KE // 16):
            s16 = s_v[pl.ds(g * 16, 16)] - coff
            d16 = d_v[pl.ds(g * 16, 16)]
            a_s = plsc.load_gather(as_v, [s16])
            a_d = plsc.load_gather(ad_v, [d16])
            t = a_s + a_d
            e = jnp.where(t >= 0.0, t, 0.2 * t)
            ex = jnp.exp(e - jnp.maximum(mxv + a_d, 0.0))
            plsc.addupdate_scatter(den_v, [d16], ex)
            e_v[pl.ds(g * 16, 16)] = ex

    def scale(r_v, e_v):
        def sbody(jj, c):
            for u in range(4):
                j = jj * 4 + u
                exj = plsc.load_gather(e_v, [jnp.zeros((16,), jnp.int32) + j])
                for cc in range(HC // 16):
                    r_v[j, pl.ds(cc * 16, 16)] = \
                        r_v[j, pl.ds(cc * 16, 16)] * exj
            return c
        lax.fori_loop(0, KE // 4, sbody, 0)

    load_idx(0, s0, d0)
    pltpu.async_copy(xw_hbm.at[s0], r0, sg0)

    def pair(i, c):
        bA = 2 * i
        pltpu.make_async_copy(xw_hbm.at[s0], r0, sg0).wait()
        compute_ex(s0, d0, e0)
        scale(r0, e0)

        @pl.when(i > 0)
        def _():
            pltpu.make_async_copy(r1, acc_sh.at[d1], ss1).wait()
        load_idx(bA + 1, s1, d1)
        pltpu.async_copy(xw_hbm.at[s1], r1, sg1)
        pltpu.async_copy(r0, acc_sh.at[d0], ss0, add=True)

        pltpu.make_async_copy(xw_hbm.at[s1], r1, sg1).wait()
        compute_ex(s1, d1, e1)
        scale(r1, e1)
        pltpu.make_async_copy(r0, acc_sh.at[d0], ss0).wait()
        load_idx(bA + 2, s0, d0)
        pltpu.async_copy(xw_hbm.at[s0], r0, sg0)
        pltpu.async_copy(r1, acc_sh.at[d1], ss1, add=True)
        return c
    lax.fori_loop(0, NB // 2 - 1, pair, 0)

    pltpu.make_async_copy(xw_hbm.at[s0], r0, sg0).wait()
    compute_ex(s0, d0, e0)
    scale(r0, e0)
    pltpu.make_async_copy(r1, acc_sh.at[d1], ss1).wait()
    load_idx(NB - 1, s1, d1)
    pltpu.async_copy(xw_hbm.at[s1], r1, sg1)
    pltpu.async_copy(r0, acc_sh.at[d0], ss0, add=True)

    pltpu.make_async_copy(xw_hbm.at[s1], r1, sg1).wait()
    compute_ex(s1, d1, e1)
    scale(r1, e1)
    pltpu.make_async_copy(r0, acc_sh.at[d0], ss0).wait()
    pltpu.sync_copy(r1, acc_sh.at[d1], add=True)

    @pl.when(cid == 0)
    def _():
        pltpu.sync_copy(den_v, den_hbm.at[sid])

    plsc.subcore_barrier()

    @pl.when(sid == 0)
    def _():
        pltpu.sync_copy(acc_sh, acc_hbm.at[cid])


def _run_edges(ei, a_s, a_d, mx16, xw2):
    f = functools.partial(
        pl.kernel,
        out_type=[
            jax.ShapeDtypeStruct((NC, N, HC), jnp.float32),
            jax.ShapeDtypeStruct((NS, N), jnp.float32),
        ],
        mesh=plsc.VectorSubcoreMesh(core_axis_name="c", subcore_axis_name="s"),
        compiler_params=pltpu.CompilerParams(needs_layout_passes=False,
                                             use_tc_tiling_on_sc=False),
        scratch_types=[
            pltpu.VMEM((N,), jnp.float32),
            pltpu.VMEM((N,), jnp.float32),
            pltpu.VMEM((N,), jnp.float32),
            pltpu.VMEM((KE,), jnp.int32),
            pltpu.VMEM((KE,), jnp.int32),
            pltpu.VMEM((KE,), jnp.int32),
            pltpu.VMEM((KE,), jnp.int32),
            pltpu.VMEM((KE, HC), jnp.float32),
            pltpu.VMEM((KE, HC), jnp.float32),
            pltpu.VMEM((KE,), jnp.float32),
            pltpu.VMEM((KE,), jnp.float32),
            pltpu.VMEM((16,), jnp.float32),
            pltpu.VMEM_SHARED((N, HC), jnp.float32),
            pltpu.SemaphoreType.DMA,
            pltpu.SemaphoreType.DMA,
            pltpu.SemaphoreType.DMA,
            pltpu.SemaphoreType.DMA,
        ],
    )(_edge_sc_body)
    return f(ei, a_s, a_d, mx16, xw2)


def _norm_body(acc_ref, den_ref, bi_ref, sid_ref, intra_ref, sec_ref):
    i = pl.program_id(0)
    den = jnp.sum(den_ref[:, i, :], axis=0) + 1e-16
    num = jnp.concatenate((acc_ref[0], acc_ref[1]), axis=1)
    out = num / den[:, None] + bi_ref[...]
    intra_ref[...] = out

    @pl.when(i == 0)
    def _():
        sec_ref[...] = jnp.full((S, H), -jnp.inf, jnp.float32)

    sid = sid_ref[...]
    cur = sec_ref[...]
    upd = []
    for s in range(S):
        mask = (sid == s)
        ms = jnp.max(jnp.where(mask, out, -jnp.inf), axis=0)
        upd.append(ms)
    sec_ref[...] = jnp.maximum(cur, jnp.stack(upd, axis=0))

    @pl.when(i == NGRID - 1)
    def _():
        fin = sec_ref[...]
        sec_ref[...] = jnp.where(jnp.isfinite(fin), fin, 0.0)


def _run_norm(acc, den, bi, sids):
    return pl.pallas_call(
        _norm_body,
        grid=(NGRID,),
        in_specs=[
            pl.BlockSpec((NC, NBLK, HC), lambda i: (0, i, 0)),
            pl.BlockSpec((NS, NGRID, NBLK), lambda i: (0, 0, 0)),
            pl.BlockSpec((1, H), lambda i: (0, 0)),
            pl.BlockSpec((NBLK, 1), lambda i: (i, 0)),
        ],
        out_specs=[
            pl.BlockSpec((NBLK, H), lambda i: (i, 0)),
            pl.BlockSpec((S, H), lambda i: (0, 0)),
        ],
        out_shape=[
            jax.ShapeDtypeStruct((N, H), jnp.float32),
            jax.ShapeDtypeStruct((S, H), jnp.float32),
        ],
    )(acc, den, bi, sids)


def _inter_body(sec_ref, we_ref, aes_ref, aed_ref, be_ref, ei_ref, wf3_ref,
                bf_ref, q_ref):
    hi = lax.Precision.HIGHEST
    sec = sec_ref[...]
    xwe = jnp.dot(sec, we_ref[...], preferred_element_type=jnp.float32,
                  precision=hi)
    als = jnp.dot(xwe, aes_ref[...], preferred_element_type=jnp.float32,
                  precision=hi)
    ald = jnp.dot(xwe, aed_ref[...], preferred_element_type=jnp.float32,
                  precision=hi)
    iot = lax.broadcasted_iota(jnp.int32, (EI, S), 1)
    srcc = ei_ref[0, :].reshape(EI, 1)
    dstc = ei_ref[1, :].reshape(EI, 1)
    oh_s = (srcc == iot).astype(jnp.float32)
    oh_d = (dstc == iot).astype(jnp.float32)
    e_als = jnp.dot(oh_s, als, preferred_element_type=jnp.float32,
                    precision=hi)
    e_ald = jnp.dot(oh_d, ald, preferred_element_type=jnp.float32,
                    precision=hi)
    t = e_als + e_ald
    e = jnp.where(t >= 0.0, t, 0.2 * t)
    m = jnp.max(jnp.where(oh_d > 0.0, e, -jnp.inf), axis=0, keepdims=True)
    m = jnp.where(jnp.isfinite(m), m, 0.0)
    md = jnp.dot(oh_d, m.reshape(S, 1), preferred_element_type=jnp.float32,
                 precision=hi)
    ex = jnp.exp(e - md)
    den = lax.dot_general(oh_d, ex, (((0,), (0,)), ((), ())),
                          preferred_element_type=jnp.float32,
                          precision=hi) + 1e-16
    dd = jnp.dot(oh_d, den, preferred_element_type=jnp.float32, precision=hi)
    alpha = ex / dd
    xs = jnp.dot(oh_s, xwe, preferred_element_type=jnp.float32, precision=hi)
    msg = alpha * xs
    inter = lax.dot_general(oh_d, msg, (((0,), (0,)), ((), ())),
                            preferred_element_type=jnp.float32,
                            precision=hi) + be_ref[...]
    q_ref[...] = jnp.dot(inter, wf3_ref[...],
                         preferred_element_type=jnp.float32,
                         precision=hi) + bf_ref[...]


def _run_inter(sec, we, aes, aed, be, ei, wf3, bf):
    return pl.pallas_call(
        _inter_body,
        out_shape=jax.ShapeDtypeStruct((S, 1), jnp.float32),
    )(sec, we, aes, aed, be, ei, wf3, bf)


def _fuse_body(seq_ref, intra_ref, sid_ref, q_ref, wf1_ref, wf2_ref, o_ref):
    hi = lax.Precision.HIGHEST
    iot = lax.broadcasted_iota(jnp.int32, (NBLK, S), 1)
    oh = (sid_ref[...] == iot).astype(jnp.float32)
    g = jnp.dot(oh, q_ref[...], preferred_element_type=jnp.float32,
                precision=hi)
    o_ref[...] = (
        jnp.dot(seq_ref[...], wf1_ref[...], preferred_element_type=jnp.float32,
                precision=hi)
        + jnp.dot(intra_ref[...], wf2_ref[...],
                  preferred_element_type=jnp.float32, precision=hi)
        + g)


def _run_fuse(seq, intra, sids, q, wf1, wf2):
    return pl.pallas_call(
        _fuse_body,
        grid=(NGRID,),
        in_specs=[
            pl.BlockSpec((NBLK, H), lambda i: (i, 0)),
            pl.BlockSpec((NBLK, H), lambda i: (i, 0)),
            pl.BlockSpec((NBLK, 1), lambda i: (i, 0)),
            pl.BlockSpec((S, 1), lambda i: (0, 0)),
            pl.BlockSpec((H, 1), lambda i: (0, 0)),
            pl.BlockSpec((H, 1), lambda i: (0, 0)),
        ],
        out_specs=pl.BlockSpec((NBLK, 1), lambda i: (i, 0)),
        out_shape=jax.ShapeDtypeStruct((N, 1), jnp.float32),
    )(seq, intra, sids, q, wf1, wf2)


@jax.jit
def kernel(x, W_ih, W_hh, b_ih, b_hh, Wi, ai_src, ai_dst, bi, We, ae_src,
           ae_dst, be, Wf, bf, intra_edge_index, inter_edge_index,
           sector_ids):
    xt = jnp.swapaxes(x, 0, 1)
    seq, xw, a_s, a_d, mx = _run_gru(
        xt, W_ih, W_hh, b_ih.reshape(1, -1), b_hh.reshape(1, -1), Wi,
        ai_src.reshape(H, 1), ai_dst.reshape(H, 1))
    mx16 = jnp.broadcast_to(mx.reshape(1), (16,))
    acc, den = _run_edges(intra_edge_index.reshape(2 * E), a_s.ravel(),
                          a_d.ravel(), mx16, xw.reshape(2 * N, HC))
    intra, sec = _run_norm(acc, den.reshape(NS, NGRID, NBLK),
                           bi.reshape(1, H), sector_ids.reshape(N, 1))
    q = _run_inter(sec, We, ae_src.reshape(H, 1), ae_dst.reshape(H, 1),
                   be.reshape(1, H), inter_edge_index, Wf[2 * H:],
                   bf.reshape(1, 1))
    out = _run_fuse(seq, intra, sector_ids.reshape(N, 1), q,
                    Wf[:H], Wf[H:2 * H])
    return out.ravel()


# speedup vs baseline: 1.2259x; 1.2259x over previous
import functools

import jax
import jax.numpy as jnp
from jax import lax
from jax.experimental import pallas as pl
from jax.experimental.pallas import tpu as pltpu
from jax.experimental.pallas import tpu_sc as plsc

N = 10000
T = 32
DIN = 16
H = 128
E = 320000
S = 64
EI = 512

NBLK = 1000
NGRID = N // NBLK

NC = 2
NS = 16
NW = NC * NS
EPT = E // NW
KE = 80
NEB = EPT // KE


def _gru_body(xt_ref, wih_ref, whh_ref, bih_ref, bhh_ref, wi_ref, ais_ref,
              aid_ref, seq_ref, xw_ref, as_ref, ad_ref, mx_ref):
    wih = wih_ref[...]
    whh = whh_ref[...]
    bih = bih_ref[...]
    bhh = bhh_ref[...]

    def step(t, h):
        xt = xt_ref[t]
        gi = jnp.dot(xt, wih, preferred_element_type=jnp.float32) + bih
        gh = jnp.dot(h, whh, preferred_element_type=jnp.float32) + bhh
        r = jax.nn.sigmoid(gi[:, :H] + gh[:, :H])
        z = jax.nn.sigmoid(gi[:, H:2 * H] + gh[:, H:2 * H])
        n = jnp.tanh(gi[:, 2 * H:] + r * gh[:, 2 * H:])
        return (1.0 - z) * n + z * h

    h = lax.fori_loop(0, T, step, jnp.zeros((NBLK, H), jnp.float32))
    seq_ref[...] = h
    xw = jnp.dot(h, wi_ref[...], preferred_element_type=jnp.float32)
    xw_ref[0] = xw[:, :H // 2]
    xw_ref[1] = xw[:, H // 2:]
    a_s = jnp.dot(xw, ais_ref[...], preferred_element_type=jnp.float32)
    a_d = jnp.dot(xw, aid_ref[...], preferred_element_type=jnp.float32)
    as_ref[...] = a_s
    ad_ref[...] = a_d
    i = pl.program_id(0)

    @pl.when(i == 0)
    def _():
        mx_ref[...] = jnp.full((1, 1), -jnp.inf, jnp.float32)

    mx_ref[...] = jnp.maximum(mx_ref[...], jnp.full((1, 1), jnp.max(a_s)))


def _run_gru(xt, w_ih, w_hh, b_ih, b_hh, wi, ai_src, ai_dst):
    return pl.pallas_call(
        _gru_body,
        grid=(NGRID,),
        in_specs=[
            pl.BlockSpec((T, NBLK, DIN), lambda i: (0, i, 0)),
            pl.BlockSpec((DIN, 3 * H), lambda i: (0, 0)),
            pl.BlockSpec((H, 3 * H), lambda i: (0, 0)),
            pl.BlockSpec((1, 3 * H), lambda i: (0, 0)),
            pl.BlockSpec((1, 3 * H), lambda i: (0, 0)),
            pl.BlockSpec((H, H), lambda i: (0, 0)),
            pl.BlockSpec((H, 1), lambda i: (0, 0)),
            pl.BlockSpec((H, 1), lambda i: (0, 0)),
        ],
        out_specs=[
            pl.BlockSpec((NBLK, H), lambda i: (i, 0)),
            pl.BlockSpec((2, NBLK, H // 2), lambda i: (0, i, 0)),
            pl.BlockSpec((NBLK, 1), lambda i: (i, 0)),
            pl.BlockSpec((NBLK, 1), lambda i: (i, 0)),
            pl.BlockSpec((1, 1), lambda i: (0, 0)),
        ],
        out_shape=[
            jax.ShapeDtypeStruct((N, H), jnp.float32),
            jax.ShapeDtypeStruct((2, N, H // 2), jnp.float32),
            jax.ShapeDtypeStruct((N, 1), jnp.float32),
            jax.ShapeDtypeStruct((N, 1), jnp.float32),
            jax.ShapeDtypeStruct((1, 1), jnp.float32),
        ],
    )(xt, w_ih, w_hh, b_ih, b_hh, wi, ai_src, ai_dst)


HC = H // 2
EPT2 = E // NS
NB = EPT2 // KE


def _edge_sc_body(ei_hbm, as_hbm, ad_hbm, mx_hbm, xw_hbm,
                  acc_hbm, den_hbm,
                  as_v, ad_v, den_v, sa_v, da_v, r0, r1, e0, e1, mx_v,
                  acc_sh, sg0, sg1, ss0, ss1):
    cid = lax.axis_index("c")
    sid = lax.axis_index("s")
    coff = cid * N

    pltpu.sync_copy(as_hbm, as_v)
    pltpu.sync_copy(ad_hbm, ad_v)
    pltpu.sync_copy(mx_hbm, mx_v)
    pltpu.sync_copy(ei_hbm.at[0, sid], sa_v)
    pltpu.sync_copy(ei_hbm.at[1, sid], da_v)
    mxv = mx_v[...]

    def soff(b, c):
        for g in range(KE // 16):
            sa_v[b, pl.ds(g * 16, 16)] = sa_v[b, pl.ds(g * 16, 16)] + coff
        return c
    lax.fori_loop(0, NB, soff, 0)

    def zden(j, c):
        den_v[pl.ds(j * 16, 16)] = jnp.zeros((16,), jnp.float32)
        return c
    lax.fori_loop(0, N // 16, zden, 0)

    def zrows(j, c):
        for cc in range(HC // 16):
            r0[j, pl.ds(cc * 16, 16)] = jnp.zeros((16,), jnp.float32)
        return c
    lax.fori_loop(0, KE, zrows, 0)

    @pl.when(sid == 0)
    def _():
        def zacc(b, c):
            pltpu.sync_copy(r0, acc_sh.at[pl.ds(b * KE, KE)])
            return c
        lax.fori_loop(0, N // KE, zacc, 0)

    plsc.subcore_barrier()

    def compute_ex(b, e_v):
        for g in range(KE // 16):
            s16 = sa_v[b, pl.ds(g * 16, 16)] - coff
            d16 = da_v[b, pl.ds(g * 16, 16)]
            a_s = plsc.load_gather(as_v, [s16])
            a_d = plsc.load_gather(ad_v, [d16])
            t = a_s + a_d
            e = jnp.where(t >= 0.0, t, 0.2 * t)
            ex = jnp.exp(e - jnp.maximum(mxv + a_d, 0.0))
            plsc.addupdate_scatter(den_v, [d16], ex)
            e_v[pl.ds(g * 16, 16)] = ex

    def scale(r_v, e_v):
        def sbody(jj, c):
            for u in range(4):
                j = jj * 4 + u
                exj = plsc.load_gather(e_v, [jnp.zeros((16,), jnp.int32) + j])
                for cc in range(HC // 16):
                    r_v[j, pl.ds(cc * 16, 16)] = \
                        r_v[j, pl.ds(cc * 16, 16)] * exj
            return c
        lax.fori_loop(0, KE // 4, sbody, 0)

    def fire_gather(b, r_v, sem):
        pltpu.async_copy(xw_hbm.at[sa_v.at[b]], r_v, sem)

    def wait_gather(b, r_v, sem):
        pltpu.make_async_copy(xw_hbm.at[sa_v.at[b]], r_v, sem).wait()

    def fire_scatter(b, r_v, sem):
        pltpu.async_copy(r_v, acc_sh.at[da_v.at[b]], sem, add=True)

    def wait_scatter(b, r_v, sem):
        pltpu.make_async_copy(r_v, acc_sh.at[da_v.at[b]], sem).wait()

    fire_gather(0, r0, sg0)

    def pair(i, c):
        bA = 2 * i
        wait_gather(bA, r0, sg0)
        compute_ex(bA, e0)
        scale(r0, e0)

        @pl.when(i > 0)
        def _():
            wait_scatter(bA - 1, r1, ss1)
        fire_gather(bA + 1, r1, sg1)
        fire_scatter(bA, r0, ss0)

        wait_gather(bA + 1, r1, sg1)
        compute_ex(bA + 1, e1)
        scale(r1, e1)
        wait_scatter(bA, r0, ss0)
        fire_gather(bA + 2, r0, sg0)
        fire_scatter(bA + 1, r1, ss1)
        return c
    lax.fori_loop(0, NB // 2 - 1, pair, 0)

    wait_gather(NB - 2, r0, sg0)
    compute_ex(NB - 2, e0)
    scale(r0, e0)
    wait_scatter(NB - 3, r1, ss1)
    fire_gather(NB - 1, r1, sg1)
    fire_scatter(NB - 2, r0, ss0)

    wait_gather(NB - 1, r1, sg1)
    compute_ex(NB - 1, e1)
    scale(r1, e1)
    wait_scatter(NB - 2, r0, ss0)
    pltpu.sync_copy(r1, acc_sh.at[da_v.at[NB - 1]], add=True)

    @pl.when(cid == 0)
    def _():
        pltpu.sync_copy(den_v, den_hbm.at[sid])

    plsc.subcore_barrier()

    @pl.when(sid == 0)
    def _():
        pltpu.sync_copy(acc_sh, acc_hbm.at[cid])


def _run_edges(ei, a_s, a_d, mx16, xw2):
    f = functools.partial(
        pl.kernel,
        out_type=[
            jax.ShapeDtypeStruct((NC, N, HC), jnp.float32),
            jax.ShapeDtypeStruct((NS, N), jnp.float32),
        ],
        mesh=plsc.VectorSubcoreMesh(core_axis_name="c", subcore_axis_name="s"),
        compiler_params=pltpu.CompilerParams(needs_layout_passes=False,
                                             use_tc_tiling_on_sc=False),
        scratch_types=[
            pltpu.VMEM((N,), jnp.float32),
            pltpu.VMEM((N,), jnp.float32),
            pltpu.VMEM((N,), jnp.float32),
            pltpu.VMEM((NB, KE), jnp.int32),
            pltpu.VMEM((NB, KE), jnp.int32),
            pltpu.VMEM((KE, HC), jnp.float32),
            pltpu.VMEM((KE, HC), jnp.float32),
            pltpu.VMEM((KE,), jnp.float32),
            pltpu.VMEM((KE,), jnp.float32),
            pltpu.VMEM((16,), jnp.float32),
            pltpu.VMEM_SHARED((N, HC), jnp.float32),
            pltpu.SemaphoreType.DMA,
            pltpu.SemaphoreType.DMA,
            pltpu.SemaphoreType.DMA,
            pltpu.SemaphoreType.DMA,
        ],
    )(_edge_sc_body)
    return f(ei, a_s, a_d, mx16, xw2)


def _norm_body(acc_ref, den_ref, bi_ref, sid_ref, intra_ref, sec_ref):
    i = pl.program_id(0)
    den = jnp.sum(den_ref[:, i, :], axis=0) + 1e-16
    num = jnp.concatenate((acc_ref[0], acc_ref[1]), axis=1)
    out = num / den[:, None] + bi_ref[...]
    intra_ref[...] = out

    @pl.when(i == 0)
    def _():
        sec_ref[...] = jnp.full((S, H), -jnp.inf, jnp.float32)

    sid = sid_ref[...]
    cur = sec_ref[...]
    upd = []
    for s in range(S):
        mask = (sid == s)
        ms = jnp.max(jnp.where(mask, out, -jnp.inf), axis=0)
        upd.append(ms)
    sec_ref[...] = jnp.maximum(cur, jnp.stack(upd, axis=0))

    @pl.when(i == NGRID - 1)
    def _():
        fin = sec_ref[...]
        sec_ref[...] = jnp.where(jnp.isfinite(fin), fin, 0.0)


def _run_norm(acc, den, bi, sids):
    return pl.pallas_call(
        _norm_body,
        grid=(NGRID,),
        in_specs=[
            pl.BlockSpec((NC, NBLK, HC), lambda i: (0, i, 0)),
            pl.BlockSpec((NS, NGRID, NBLK), lambda i: (0, 0, 0)),
            pl.BlockSpec((1, H), lambda i: (0, 0)),
            pl.BlockSpec((NBLK, 1), lambda i: (i, 0)),
        ],
        out_specs=[
            pl.BlockSpec((NBLK, H), lambda i: (i, 0)),
            pl.BlockSpec((S, H), lambda i: (0, 0)),
        ],
        out_shape=[
            jax.ShapeDtypeStruct((N, H), jnp.float32),
            jax.ShapeDtypeStruct((S, H), jnp.float32),
        ],
    )(acc, den, bi, sids)


def _inter_body(sec_ref, we_ref, aes_ref, aed_ref, be_ref, ei_ref, wf3_ref,
                bf_ref, q_ref):
    hi = lax.Precision.HIGHEST
    sec = sec_ref[...]
    xwe = jnp.dot(sec, we_ref[...], preferred_element_type=jnp.float32,
                  precision=hi)
    als = jnp.dot(xwe, aes_ref[...], preferred_element_type=jnp.float32,
                  precision=hi)
    ald = jnp.dot(xwe, aed_ref[...], preferred_element_type=jnp.float32,
                  precision=hi)
    iot = lax.broadcasted_iota(jnp.int32, (EI, S), 1)
    srcc = ei_ref[0, :].reshape(EI, 1)
    dstc = ei_ref[1, :].reshape(EI, 1)
    oh_s = (srcc == iot).astype(jnp.float32)
    oh_d = (dstc == iot).astype(jnp.float32)
    e_als = jnp.dot(oh_s, als, preferred_element_type=jnp.float32,
                    precision=hi)
    e_ald = jnp.dot(oh_d, ald, preferred_element_type=jnp.float32,
                    precision=hi)
    t = e_als + e_ald
    e = jnp.where(t >= 0.0, t, 0.2 * t)
    m = jnp.max(jnp.where(oh_d > 0.0, e, -jnp.inf), axis=0, keepdims=True)
    m = jnp.where(jnp.isfinite(m), m, 0.0)
    md = jnp.dot(oh_d, m.reshape(S, 1), preferred_element_type=jnp.float32,
                 precision=hi)
    ex = jnp.exp(e - md)
    den = lax.dot_general(oh_d, ex, (((0,), (0,)), ((), ())),
                          preferred_element_type=jnp.float32,
                          precision=hi) + 1e-16
    dd = jnp.dot(oh_d, den, preferred_element_type=jnp.float32, precision=hi)
    alpha = ex / dd
    xs = jnp.dot(oh_s, xwe, preferred_element_type=jnp.float32, precision=hi)
    msg = alpha * xs
    inter = lax.dot_general(oh_d, msg, (((0,), (0,)), ((), ())),
                            preferred_element_type=jnp.float32,
                            precision=hi) + be_ref[...]
    q_ref[...] = jnp.dot(inter, wf3_ref[...],
                         preferred_element_type=jnp.float32,
                         precision=hi) + bf_ref[...]


def _run_inter(sec, we, aes, aed, be, ei, wf3, bf):
    return pl.pallas_call(
        _inter_body,
        out_shape=jax.ShapeDtypeStruct((S, 1), jnp.float32),
    )(sec, we, aes, aed, be, ei, wf3, bf)


def _fuse_body(seq_ref, intra_ref, sid_ref, q_ref, wf1_ref, wf2_ref, o_ref):
    hi = lax.Precision.HIGHEST
    iot = lax.broadcasted_iota(jnp.int32, (NBLK, S), 1)
    oh = (sid_ref[...] == iot).astype(jnp.float32)
    g = jnp.dot(oh, q_ref[...], preferred_element_type=jnp.float32,
                precision=hi)
    o_ref[...] = (
        jnp.dot(seq_ref[...], wf1_ref[...], preferred_element_type=jnp.float32,
                precision=hi)
        + jnp.dot(intra_ref[...], wf2_ref[...],
                  preferred_element_type=jnp.float32, precision=hi)
        + g)


def _run_fuse(seq, intra, sids, q, wf1, wf2):
    return pl.pallas_call(
        _fuse_body,
        grid=(NGRID,),
        in_specs=[
            pl.BlockSpec((NBLK, H), lambda i: (i, 0)),
            pl.BlockSpec((NBLK, H), lambda i: (i, 0)),
            pl.BlockSpec((NBLK, 1), lambda i: (i, 0)),
            pl.BlockSpec((S, 1), lambda i: (0, 0)),
            pl.BlockSpec((H, 1), lambda i: (0, 0)),
            pl.BlockSpec((H, 1), lambda i: (0, 0)),
        ],
        out_specs=pl.BlockSpec((NBLK, 1), lambda i: (i, 0)),
        out_shape=jax.ShapeDtypeStruct((N, 1), jnp.float32),
    )(seq, intra, sids, q, wf1, wf2)


@jax.jit
def kernel(x, W_ih, W_hh, b_ih, b_hh, Wi, ai_src, ai_dst, bi, We, ae_src,
           ae_dst, be, Wf, bf, intra_edge_index, inter_edge_index,
           sector_ids):
    xt = jnp.swapaxes(x, 0, 1)
    seq, xw, a_s, a_d, mx = _run_gru(
        xt, W_ih, W_hh, b_ih.reshape(1, -1), b_hh.reshape(1, -1), Wi,
        ai_src.reshape(H, 1), ai_dst.reshape(H, 1))
    mx16 = jnp.broadcast_to(mx.reshape(1), (16,))
    acc, den = _run_edges(intra_edge_index.reshape(2, NS, NB, KE),
                          a_s.ravel(), a_d.ravel(), mx16,
                          xw.reshape(2 * N, HC))
    intra, sec = _run_norm(acc, den.reshape(NS, NGRID, NBLK),
                           bi.reshape(1, H), sector_ids.reshape(N, 1))
    q = _run_inter(sec, We, ae_src.reshape(H, 1), ae_dst.reshape(H, 1),
                   be.reshape(1, H), inter_edge_index, Wf[2 * H:],
                   bf.reshape(1, 1))
    out = _run_fuse(seq, intra, sector_ids.reshape(N, 1), q,
                    Wf[:H], Wf[H:2 * H])
    return out.ravel()


# speedup vs baseline: 1.4458x; 1.1793x over previous
import functools

import jax
import jax.numpy as jnp
from jax import lax
from jax.experimental import pallas as pl
from jax.experimental.pallas import tpu as pltpu
from jax.experimental.pallas import tpu_sc as plsc

N = 10000
T = 32
DIN = 16
H = 128
E = 320000
S = 64
EI = 512

NBLK = 1000
NGRID = N // NBLK

NC = 2
NS = 16
NW = NC * NS
EPT = E // NW
KE = 80
NEB = EPT // KE


def _gru_body(xt_ref, wih_ref, whh_ref, bih_ref, bhh_ref, wi_ref, ais_ref,
              aid_ref, seq_ref, xw_ref, as_ref, ad_ref, mx_ref):
    wih = wih_ref[...]
    whh = whh_ref[...]
    bih = bih_ref[...]
    bhh = bhh_ref[...]

    def step(t, h):
        xt = xt_ref[t]
        gi = jnp.dot(xt, wih, preferred_element_type=jnp.float32) + bih
        gh = jnp.dot(h, whh, preferred_element_type=jnp.float32) + bhh
        r = jax.nn.sigmoid(gi[:, :H] + gh[:, :H])
        z = jax.nn.sigmoid(gi[:, H:2 * H] + gh[:, H:2 * H])
        n = jnp.tanh(gi[:, 2 * H:] + r * gh[:, 2 * H:])
        return (1.0 - z) * n + z * h

    h = lax.fori_loop(0, T, step, jnp.zeros((NBLK, H), jnp.float32))
    seq_ref[...] = h
    xw = jnp.dot(h, wi_ref[...], preferred_element_type=jnp.float32)
    xw_ref[0] = xw[:, :H // 2]
    xw_ref[1] = xw[:, H // 2:]
    a_s = jnp.dot(xw, ais_ref[...], preferred_element_type=jnp.float32)
    a_d = jnp.dot(xw, aid_ref[...], preferred_element_type=jnp.float32)
    as_ref[...] = a_s
    ad_ref[...] = a_d
    i = pl.program_id(0)

    @pl.when(i == 0)
    def _():
        mx_ref[...] = jnp.full((1, 1), -jnp.inf, jnp.float32)

    mx_ref[...] = jnp.maximum(mx_ref[...], jnp.full((1, 1), jnp.max(a_s)))


def _run_gru(xt, w_ih, w_hh, b_ih, b_hh, wi, ai_src, ai_dst):
    return pl.pallas_call(
        _gru_body,
        grid=(NGRID,),
        in_specs=[
            pl.BlockSpec((T, NBLK, DIN), lambda i: (0, i, 0)),
            pl.BlockSpec((DIN, 3 * H), lambda i: (0, 0)),
            pl.BlockSpec((H, 3 * H), lambda i: (0, 0)),
            pl.BlockSpec((1, 3 * H), lambda i: (0, 0)),
            pl.BlockSpec((1, 3 * H), lambda i: (0, 0)),
            pl.BlockSpec((H, H), lambda i: (0, 0)),
            pl.BlockSpec((H, 1), lambda i: (0, 0)),
            pl.BlockSpec((H, 1), lambda i: (0, 0)),
        ],
        out_specs=[
            pl.BlockSpec((NBLK, H), lambda i: (i, 0)),
            pl.BlockSpec((2, NBLK, H // 2), lambda i: (0, i, 0)),
            pl.BlockSpec((NBLK, 1), lambda i: (i, 0)),
            pl.BlockSpec((NBLK, 1), lambda i: (i, 0)),
            pl.BlockSpec((1, 1), lambda i: (0, 0)),
        ],
        out_shape=[
            jax.ShapeDtypeStruct((N, H), jnp.float32),
            jax.ShapeDtypeStruct((2, N, H // 2), jnp.float32),
            jax.ShapeDtypeStruct((N, 1), jnp.float32),
            jax.ShapeDtypeStruct((N, 1), jnp.float32),
            jax.ShapeDtypeStruct((1, 1), jnp.float32),
        ],
    )(xt, w_ih, w_hh, b_ih, b_hh, wi, ai_src, ai_dst)


HC = H // 2
EPT2 = E // NS
NB = EPT2 // KE


def _edge_sc_body(ei_hbm, as_hbm, ad_hbm, mx_hbm, xw_hbm,
                  acc_hbm, den_hbm,
                  as_v, ad_v, den_v, sa_v, da_v, r0, r1, e0, e1, mx_v,
                  acc_sh, sg0, sg1, ss0, ss1):
    cid = lax.axis_index("c")
    sid = lax.axis_index("s")
    coff = cid * N

    pltpu.sync_copy(as_hbm, as_v)
    pltpu.sync_copy(ad_hbm, ad_v)
    pltpu.sync_copy(mx_hbm, mx_v)
    pltpu.sync_copy(ei_hbm.at[0, sid], sa_v)
    pltpu.sync_copy(ei_hbm.at[1, sid], da_v)
    mxv = mx_v[...]

    def soff(b, c):
        for g in range(KE // 16):
            sa_v[b, pl.ds(g * 16, 16)] = sa_v[b, pl.ds(g * 16, 16)] + coff
        return c
    lax.fori_loop(0, NB, soff, 0)

    def zden(j, c):
        den_v[pl.ds(j * 16, 16)] = jnp.zeros((16,), jnp.float32)
        return c
    lax.fori_loop(0, N // 16, zden, 0)

    def zrows(j, c):
        for cc in range(HC // 16):
            r0[j, pl.ds(cc * 16, 16)] = jnp.zeros((16,), jnp.float32)
        return c
    lax.fori_loop(0, KE, zrows, 0)

    @pl.when(sid == 0)
    def _():
        def zacc(b, c):
            pltpu.sync_copy(r0, acc_sh.at[pl.ds(b * KE, KE)])
            return c
        lax.fori_loop(0, N // KE, zacc, 0)

    plsc.subcore_barrier()

    def compute_ex(b, e_v):
        for g in range(KE // 16):
            s16 = sa_v[b, pl.ds(g * 16, 16)] - coff
            d16 = da_v[b, pl.ds(g * 16, 16)]
            a_s = plsc.load_gather(as_v, [s16])
            a_d = plsc.load_gather(ad_v, [d16])
            t = a_s + a_d
            e = jnp.where(t >= 0.0, t, 0.2 * t)
            ex = jnp.exp(e - jnp.maximum(mxv + a_d, 0.0))
            plsc.addupdate_scatter(den_v, [d16], ex)
            e_v[pl.ds(g * 16, 16)] = ex

    def scale(r_v, e_v):
        def sbody(jj, c):
            for u in range(4):
                j = jj * 4 + u
                exj = plsc.load_gather(e_v, [jnp.zeros((16,), jnp.int32) + j])
                for cc in range(HC // 16):
                    r_v[j, pl.ds(cc * 16, 16)] = \
                        r_v[j, pl.ds(cc * 16, 16)] * exj
            return c
        lax.fori_loop(0, KE // 4, sbody, 0)

    def fire_gather(b, r_v, sem):
        pltpu.async_copy(xw_hbm.at[sa_v.at[b]], r_v, sem)

    def wait_gather(b, r_v, sem):
        pltpu.make_async_copy(xw_hbm.at[sa_v.at[b]], r_v, sem).wait()

    def fire_scatter(b, r_v, sem):
        pltpu.async_copy(r_v, acc_sh.at[da_v.at[b]], sem, add=True)

    def wait_scatter(b, r_v, sem):
        pltpu.make_async_copy(r_v, acc_sh.at[da_v.at[b]], sem).wait()

    fire_gather(0, r0, sg0)

    def pair(i, c):
        bA = 2 * i
        wait_gather(bA, r0, sg0)
        compute_ex(bA, e0)

        @pl.when(i > 0)
        def _():
            wait_scatter(bA - 1, r1, ss1)
        fire_gather(bA + 1, r1, sg1)
        scale(r0, e0)
        fire_scatter(bA, r0, ss0)

        wait_gather(bA + 1, r1, sg1)
        compute_ex(bA + 1, e1)
        wait_scatter(bA, r0, ss0)
        fire_gather(bA + 2, r0, sg0)
        scale(r1, e1)
        fire_scatter(bA + 1, r1, ss1)
        return c
    lax.fori_loop(0, NB // 2 - 1, pair, 0)

    wait_gather(NB - 2, r0, sg0)
    compute_ex(NB - 2, e0)
    wait_scatter(NB - 3, r1, ss1)
    fire_gather(NB - 1, r1, sg1)
    scale(r0, e0)
    fire_scatter(NB - 2, r0, ss0)

    wait_gather(NB - 1, r1, sg1)
    compute_ex(NB - 1, e1)
    scale(r1, e1)
    wait_scatter(NB - 2, r0, ss0)
    pltpu.sync_copy(r1, acc_sh.at[da_v.at[NB - 1]], add=True)

    @pl.when(cid == 0)
    def _():
        pltpu.sync_copy(den_v, den_hbm.at[sid])

    plsc.subcore_barrier()

    @pl.when(sid == 0)
    def _():
        pltpu.sync_copy(acc_sh, acc_hbm.at[cid])


def _run_edges(ei, a_s, a_d, mx16, xw2):
    f = functools.partial(
        pl.kernel,
        out_type=[
            jax.ShapeDtypeStruct((NC, N, HC), jnp.float32),
            jax.ShapeDtypeStruct((NS, N), jnp.float32),
        ],
        mesh=plsc.VectorSubcoreMesh(core_axis_name="c", subcore_axis_name="s"),
        compiler_params=pltpu.CompilerParams(needs_layout_passes=False,
                                             use_tc_tiling_on_sc=False),
        scratch_types=[
            pltpu.VMEM((N,), jnp.float32),
            pltpu.VMEM((N,), jnp.float32),
            pltpu.VMEM((N,), jnp.float32),
            pltpu.VMEM((NB, KE), jnp.int32),
            pltpu.VMEM((NB, KE), jnp.int32),
            pltpu.VMEM((KE, HC), jnp.float32),
            pltpu.VMEM((KE, HC), jnp.float32),
            pltpu.VMEM((KE,), jnp.float32),
            pltpu.VMEM((KE,), jnp.float32),
            pltpu.VMEM((16,), jnp.float32),
            pltpu.VMEM_SHARED((N, HC), jnp.float32),
            pltpu.SemaphoreType.DMA,
            pltpu.SemaphoreType.DMA,
            pltpu.SemaphoreType.DMA,
            pltpu.SemaphoreType.DMA,
        ],
    )(_edge_sc_body)
    return f(ei, a_s, a_d, mx16, xw2)


def _norm_body(acc_ref, den_ref, bi_ref, sid_ref, intra_ref, sec_ref):
    i = pl.program_id(0)
    den = jnp.sum(den_ref[:, i, :], axis=0) + 1e-16
    num = jnp.concatenate((acc_ref[0], acc_ref[1]), axis=1)
    out = num / den[:, None] + bi_ref[...]
    intra_ref[...] = out

    @pl.when(i == 0)
    def _():
        sec_ref[...] = jnp.full((S, H), -jnp.inf, jnp.float32)

    sid = sid_ref[...]
    cur = sec_ref[...]
    upd = []
    for s in range(S):
        mask = (sid == s)
        ms = jnp.max(jnp.where(mask, out, -jnp.inf), axis=0)
        upd.append(ms)
    sec_ref[...] = jnp.maximum(cur, jnp.stack(upd, axis=0))

    @pl.when(i == NGRID - 1)
    def _():
        fin = sec_ref[...]
        sec_ref[...] = jnp.where(jnp.isfinite(fin), fin, 0.0)


def _run_norm(acc, den, bi, sids):
    return pl.pallas_call(
        _norm_body,
        grid=(NGRID,),
        in_specs=[
            pl.BlockSpec((NC, NBLK, HC), lambda i: (0, i, 0)),
            pl.BlockSpec((NS, NGRID, NBLK), lambda i: (0, 0, 0)),
            pl.BlockSpec((1, H), lambda i: (0, 0)),
            pl.BlockSpec((NBLK, 1), lambda i: (i, 0)),
        ],
        out_specs=[
            pl.BlockSpec((NBLK, H), lambda i: (i, 0)),
            pl.BlockSpec((S, H), lambda i: (0, 0)),
        ],
        out_shape=[
            jax.ShapeDtypeStruct((N, H), jnp.float32),
            jax.ShapeDtypeStruct((S, H), jnp.float32),
        ],
    )(acc, den, bi, sids)


def _inter_body(sec_ref, we_ref, aes_ref, aed_ref, be_ref, ei_ref, wf3_ref,
                bf_ref, q_ref):
    hi = lax.Precision.HIGHEST
    sec = sec_ref[...]
    xwe = jnp.dot(sec, we_ref[...], preferred_element_type=jnp.float32,
                  precision=hi)
    als = jnp.dot(xwe, aes_ref[...], preferred_element_type=jnp.float32,
                  precision=hi)
    ald = jnp.dot(xwe, aed_ref[...], preferred_element_type=jnp.float32,
                  precision=hi)
    iot = lax.broadcasted_iota(jnp.int32, (EI, S), 1)
    srcc = ei_ref[0, :].reshape(EI, 1)
    dstc = ei_ref[1, :].reshape(EI, 1)
    oh_s = (srcc == iot).astype(jnp.float32)
    oh_d = (dstc == iot).astype(jnp.float32)
    e_als = jnp.dot(oh_s, als, preferred_element_type=jnp.float32,
                    precision=hi)
    e_ald = jnp.dot(oh_d, ald, preferred_element_type=jnp.float32,
                    precision=hi)
    t = e_als + e_ald
    e = jnp.where(t >= 0.0, t, 0.2 * t)
    m = jnp.max(jnp.where(oh_d > 0.0, e, -jnp.inf), axis=0, keepdims=True)
    m = jnp.where(jnp.isfinite(m), m, 0.0)
    md = jnp.dot(oh_d, m.reshape(S, 1), preferred_element_type=jnp.float32,
                 precision=hi)
    ex = jnp.exp(e - md)
    den = lax.dot_general(oh_d, ex, (((0,), (0,)), ((), ())),
                          preferred_element_type=jnp.float32,
                          precision=hi) + 1e-16
    dd = jnp.dot(oh_d, den, preferred_element_type=jnp.float32, precision=hi)
    alpha = ex / dd
    xs = jnp.dot(oh_s, xwe, preferred_element_type=jnp.float32, precision=hi)
    msg = alpha * xs
    inter = lax.dot_general(oh_d, msg, (((0,), (0,)), ((), ())),
                            preferred_element_type=jnp.float32,
                            precision=hi) + be_ref[...]
    q_ref[...] = jnp.dot(inter, wf3_ref[...],
                         preferred_element_type=jnp.float32,
                         precision=hi) + bf_ref[...]


def _run_inter(sec, we, aes, aed, be, ei, wf3, bf):
    return pl.pallas_call(
        _inter_body,
        out_shape=jax.ShapeDtypeStruct((S, 1), jnp.float32),
    )(sec, we, aes, aed, be, ei, wf3, bf)


def _fuse_body(seq_ref, intra_ref, sid_ref, q_ref, wf1_ref, wf2_ref, o_ref):
    hi = lax.Precision.HIGHEST
    iot = lax.broadcasted_iota(jnp.int32, (NBLK, S), 1)
    oh = (sid_ref[...] == iot).astype(jnp.float32)
    g = jnp.dot(oh, q_ref[...], preferred_element_type=jnp.float32,
                precision=hi)
    o_ref[...] = (
        jnp.dot(seq_ref[...], wf1_ref[...], preferred_element_type=jnp.float32,
                precision=hi)
        + jnp.dot(intra_ref[...], wf2_ref[...],
                  preferred_element_type=jnp.float32, precision=hi)
        + g)


def _run_fuse(seq, intra, sids, q, wf1, wf2):
    return pl.pallas_call(
        _fuse_body,
        grid=(NGRID,),
        in_specs=[
            pl.BlockSpec((NBLK, H), lambda i: (i, 0)),
            pl.BlockSpec((NBLK, H), lambda i: (i, 0)),
            pl.BlockSpec((NBLK, 1), lambda i: (i, 0)),
            pl.BlockSpec((S, 1), lambda i: (0, 0)),
            pl.BlockSpec((H, 1), lambda i: (0, 0)),
            pl.BlockSpec((H, 1), lambda i: (0, 0)),
        ],
        out_specs=pl.BlockSpec((NBLK, 1), lambda i: (i, 0)),
        out_shape=jax.ShapeDtypeStruct((N, 1), jnp.float32),
    )(seq, intra, sids, q, wf1, wf2)


@jax.jit
def kernel(x, W_ih, W_hh, b_ih, b_hh, Wi, ai_src, ai_dst, bi, We, ae_src,
           ae_dst, be, Wf, bf, intra_edge_index, inter_edge_index,
           sector_ids):
    xt = jnp.swapaxes(x, 0, 1)
    seq, xw, a_s, a_d, mx = _run_gru(
        xt, W_ih, W_hh, b_ih.reshape(1, -1), b_hh.reshape(1, -1), Wi,
        ai_src.reshape(H, 1), ai_dst.reshape(H, 1))
    mx16 = jnp.broadcast_to(mx.reshape(1), (16,))
    acc, den = _run_edges(intra_edge_index.reshape(2, NS, NB, KE),
                          a_s.ravel(), a_d.ravel(), mx16,
                          xw.reshape(2 * N, HC))
    intra, sec = _run_norm(acc, den.reshape(NS, NGRID, NBLK),
                           bi.reshape(1, H), sector_ids.reshape(N, 1))
    q = _run_inter(sec, We, ae_src.reshape(H, 1), ae_dst.reshape(H, 1),
                   be.reshape(1, H), inter_edge_index, Wf[2 * H:],
                   bf.reshape(1, 1))
    out = _run_fuse(seq, intra, sector_ids.reshape(N, 1), q,
                    Wf[:H], Wf[H:2 * H])
    return out.ravel()


# speedup vs baseline: 1.5302x; 1.0584x over previous
import functools

import jax
import jax.numpy as jnp
from jax import lax
from jax.experimental import pallas as pl
from jax.experimental.pallas import tpu as pltpu
from jax.experimental.pallas import tpu_sc as plsc

N = 10000
T = 32
DIN = 16
H = 128
E = 320000
S = 64
EI = 512

NBLK = 1000
NGRID = N // NBLK

NC = 2
NS = 16
NW = NC * NS
EPT = E // NW
KE = 80
NEB = EPT // KE


def _gru_body(xt_ref, wih_ref, whh_ref, bih_ref, bhh_ref, wi_ref, ais_ref,
              aid_ref, seq_ref, xw_ref, as_ref, ad_ref, mx_ref):
    wih = wih_ref[...]
    whh = whh_ref[...]
    bih = bih_ref[...]
    bhh = bhh_ref[...]
    w_rz = jnp.concatenate((wih[:, :2 * H], whh[:, :2 * H]), axis=0)
    b_rz = bih[:, :2 * H] + bhh[:, :2 * H]
    wi_n = wih[:, 2 * H:]
    wh_n = whh[:, 2 * H:]
    bi_n = bih[:, 2 * H:]
    bh_n = bhh[:, 2 * H:]

    def step(t, h):
        xt = xt_ref[t]
        xh = jnp.concatenate((xt, h), axis=1)
        g_rz = jnp.dot(xh, w_rz, preferred_element_type=jnp.float32) + b_rz
        r = jax.nn.sigmoid(g_rz[:, :H])
        z = jax.nn.sigmoid(g_rz[:, H:])
        gin = jnp.dot(xt, wi_n, preferred_element_type=jnp.float32) + bi_n
        ghn = jnp.dot(h, wh_n, preferred_element_type=jnp.float32) + bh_n
        n = jnp.tanh(gin + r * ghn)
        return (1.0 - z) * n + z * h

    h = lax.fori_loop(0, T, step, jnp.zeros((NBLK, H), jnp.float32))
    seq_ref[...] = h
    xw = jnp.dot(h, wi_ref[...], preferred_element_type=jnp.float32)
    xw_ref[0] = xw[:, :H // 2]
    xw_ref[1] = xw[:, H // 2:]
    a_s = jnp.dot(xw, ais_ref[...], preferred_element_type=jnp.float32)
    a_d = jnp.dot(xw, aid_ref[...], preferred_element_type=jnp.float32)
    as_ref[...] = a_s
    ad_ref[...] = a_d
    i = pl.program_id(0)

    @pl.when(i == 0)
    def _():
        mx_ref[...] = jnp.full((1, 1), -jnp.inf, jnp.float32)

    mx_ref[...] = jnp.maximum(mx_ref[...], jnp.full((1, 1), jnp.max(a_s)))


def _run_gru(xt, w_ih, w_hh, b_ih, b_hh, wi, ai_src, ai_dst):
    return pl.pallas_call(
        _gru_body,
        grid=(NGRID,),
        in_specs=[
            pl.BlockSpec((T, NBLK, DIN), lambda i: (0, i, 0)),
            pl.BlockSpec((DIN, 3 * H), lambda i: (0, 0)),
            pl.BlockSpec((H, 3 * H), lambda i: (0, 0)),
            pl.BlockSpec((1, 3 * H), lambda i: (0, 0)),
            pl.BlockSpec((1, 3 * H), lambda i: (0, 0)),
            pl.BlockSpec((H, H), lambda i: (0, 0)),
            pl.BlockSpec((H, 1), lambda i: (0, 0)),
            pl.BlockSpec((H, 1), lambda i: (0, 0)),
        ],
        out_specs=[
            pl.BlockSpec((NBLK, H), lambda i: (i, 0)),
            pl.BlockSpec((2, NBLK, H // 2), lambda i: (0, i, 0)),
            pl.BlockSpec((NBLK, 1), lambda i: (i, 0)),
            pl.BlockSpec((NBLK, 1), lambda i: (i, 0)),
            pl.BlockSpec((1, 1), lambda i: (0, 0)),
        ],
        out_shape=[
            jax.ShapeDtypeStruct((N, H), jnp.float32),
            jax.ShapeDtypeStruct((2, N, H // 2), jnp.float32),
            jax.ShapeDtypeStruct((N, 1), jnp.float32),
            jax.ShapeDtypeStruct((N, 1), jnp.float32),
            jax.ShapeDtypeStruct((1, 1), jnp.float32),
        ],
    )(xt, w_ih, w_hh, b_ih, b_hh, wi, ai_src, ai_dst)


HC = H // 2
EPT2 = E // NS
NB = EPT2 // KE


def _edge_sc_body(ei_hbm, as_hbm, ad_hbm, mx_hbm, xw_hbm,
                  acc_hbm, den_hbm,
                  as_v, ad_v, den_v, sa_v, da_v, r0, r1, e0, e1, mx_v,
                  acc_sh, sg0, sg1, ss0, ss1):
    cid = lax.axis_index("c")
    sid = lax.axis_index("s")
    coff = cid * N

    pltpu.sync_copy(as_hbm, as_v)
    pltpu.sync_copy(ad_hbm, ad_v)
    pltpu.sync_copy(mx_hbm, mx_v)
    pltpu.sync_copy(ei_hbm.at[0, sid], sa_v)
    pltpu.sync_copy(ei_hbm.at[1, sid], da_v)
    mxv = mx_v[...]

    def soff(b, c):
        for g in range(KE // 16):
            sa_v[b, pl.ds(g * 16, 16)] = sa_v[b, pl.ds(g * 16, 16)] + coff
        return c
    lax.fori_loop(0, NB, soff, 0)

    def zden(j, c):
        den_v[pl.ds(j * 16, 16)] = jnp.zeros((16,), jnp.float32)
        return c
    lax.fori_loop(0, N // 16, zden, 0)

    def zrows(j, c):
        for cc in range(HC // 16):
            r0[j, pl.ds(cc * 16, 16)] = jnp.zeros((16,), jnp.float32)
        return c
    lax.fori_loop(0, KE, zrows, 0)

    @pl.when(sid == 0)
    def _():
        def zacc(b, c):
            pltpu.sync_copy(r0, acc_sh.at[pl.ds(b * KE, KE)])
            return c
        lax.fori_loop(0, N // KE, zacc, 0)

    plsc.subcore_barrier()

    def compute_ex(b, e_v):
        for g in range(KE // 16):
            s16 = sa_v[b, pl.ds(g * 16, 16)] - coff
            d16 = da_v[b, pl.ds(g * 16, 16)]
            a_s = plsc.load_gather(as_v, [s16])
            a_d = plsc.load_gather(ad_v, [d16])
            t = a_s + a_d
            e = jnp.where(t >= 0.0, t, 0.2 * t)
            ex = jnp.exp(e - jnp.maximum(mxv + a_d, 0.0))
            plsc.addupdate_scatter(den_v, [d16], ex)
            e_v[pl.ds(g * 16, 16)] = ex

    def scale(r_v, e_v):
        def sbody(jj, c):
            for u in range(4):
                j = jj * 4 + u
                exj = plsc.load_gather(e_v, [jnp.zeros((16,), jnp.int32) + j])
                for cc in range(HC // 16):
                    r_v[j, pl.ds(cc * 16, 16)] = \
                        r_v[j, pl.ds(cc * 16, 16)] * exj
            return c
        lax.fori_loop(0, KE // 4, sbody, 0)

    def fire_gather(b, r_v, sem):
        pltpu.async_copy(xw_hbm.at[sa_v.at[b]], r_v, sem)

    def wait_gather(b, r_v, sem):
        pltpu.make_async_copy(xw_hbm.at[sa_v.at[b]], r_v, sem).wait()

    def fire_scatter(b, r_v, sem):
        pltpu.async_copy(r_v, acc_sh.at[da_v.at[b]], sem, add=True)

    def wait_scatter(b, r_v, sem):
        pltpu.make_async_copy(r_v, acc_sh.at[da_v.at[b]], sem).wait()

    fire_gather(0, r0, sg0)

    def pair(i, c):
        bA = 2 * i
        wait_gather(bA, r0, sg0)
        compute_ex(bA, e0)

        @pl.when(i > 0)
        def _():
            wait_scatter(bA - 1, r1, ss1)
        fire_gather(bA + 1, r1, sg1)
        scale(r0, e0)
        fire_scatter(bA, r0, ss0)

        wait_gather(bA + 1, r1, sg1)
        compute_ex(bA + 1, e1)
        wait_scatter(bA, r0, ss0)
        fire_gather(bA + 2, r0, sg0)
        scale(r1, e1)
        fire_scatter(bA + 1, r1, ss1)
        return c
    lax.fori_loop(0, NB // 2 - 1, pair, 0)

    wait_gather(NB - 2, r0, sg0)
    compute_ex(NB - 2, e0)
    wait_scatter(NB - 3, r1, ss1)
    fire_gather(NB - 1, r1, sg1)
    scale(r0, e0)
    fire_scatter(NB - 2, r0, ss0)

    wait_gather(NB - 1, r1, sg1)
    compute_ex(NB - 1, e1)
    scale(r1, e1)
    wait_scatter(NB - 2, r0, ss0)
    pltpu.sync_copy(r1, acc_sh.at[da_v.at[NB - 1]], add=True)

    @pl.when(cid == 0)
    def _():
        pltpu.sync_copy(den_v, den_hbm.at[sid])

    plsc.subcore_barrier()

    @pl.when(sid == 0)
    def _():
        pltpu.sync_copy(acc_sh, acc_hbm.at[cid])


def _run_edges(ei, a_s, a_d, mx16, xw2):
    f = functools.partial(
        pl.kernel,
        out_type=[
            jax.ShapeDtypeStruct((NC, N, HC), jnp.float32),
            jax.ShapeDtypeStruct((NS, N), jnp.float32),
        ],
        mesh=plsc.VectorSubcoreMesh(core_axis_name="c", subcore_axis_name="s"),
        compiler_params=pltpu.CompilerParams(needs_layout_passes=False,
                                             use_tc_tiling_on_sc=False),
        scratch_types=[
            pltpu.VMEM((N,), jnp.float32),
            pltpu.VMEM((N,), jnp.float32),
            pltpu.VMEM((N,), jnp.float32),
            pltpu.VMEM((NB, KE), jnp.int32),
            pltpu.VMEM((NB, KE), jnp.int32),
            pltpu.VMEM((KE, HC), jnp.float32),
            pltpu.VMEM((KE, HC), jnp.float32),
            pltpu.VMEM((KE,), jnp.float32),
            pltpu.VMEM((KE,), jnp.float32),
            pltpu.VMEM((16,), jnp.float32),
            pltpu.VMEM_SHARED((N, HC), jnp.float32),
            pltpu.SemaphoreType.DMA,
            pltpu.SemaphoreType.DMA,
            pltpu.SemaphoreType.DMA,
            pltpu.SemaphoreType.DMA,
        ],
    )(_edge_sc_body)
    return f(ei, a_s, a_d, mx16, xw2)


def _norm_body(acc_ref, den_ref, bi_ref, sid_ref, intra_ref, sec_ref):
    i = pl.program_id(0)
    den = jnp.sum(den_ref[:, i, :], axis=0) + 1e-16
    num = jnp.concatenate((acc_ref[0], acc_ref[1]), axis=1)
    out = num / den[:, None] + bi_ref[...]
    intra_ref[...] = out

    @pl.when(i == 0)
    def _():
        sec_ref[...] = jnp.full((S, H), -jnp.inf, jnp.float32)

    sid = sid_ref[...]
    cur = sec_ref[...]
    upd = []
    for s in range(S):
        mask = (sid == s)
        ms = jnp.max(jnp.where(mask, out, -jnp.inf), axis=0)
        upd.append(ms)
    sec_ref[...] = jnp.maximum(cur, jnp.stack(upd, axis=0))

    @pl.when(i == NGRID - 1)
    def _():
        fin = sec_ref[...]
        sec_ref[...] = jnp.where(jnp.isfinite(fin), fin, 0.0)


def _run_norm(acc, den, bi, sids):
    return pl.pallas_call(
        _norm_body,
        grid=(NGRID,),
        in_specs=[
            pl.BlockSpec((NC, NBLK, HC), lambda i: (0, i, 0)),
            pl.BlockSpec((NS, NGRID, NBLK), lambda i: (0, 0, 0)),
            pl.BlockSpec((1, H), lambda i: (0, 0)),
            pl.BlockSpec((NBLK, 1), lambda i: (i, 0)),
        ],
        out_specs=[
            pl.BlockSpec((NBLK, H), lambda i: (i, 0)),
            pl.BlockSpec((S, H), lambda i: (0, 0)),
        ],
        out_shape=[
            jax.ShapeDtypeStruct((N, H), jnp.float32),
            jax.ShapeDtypeStruct((S, H), jnp.float32),
        ],
    )(acc, den, bi, sids)


def _inter_body(sec_ref, we_ref, aes_ref, aed_ref, be_ref, ei_ref, wf3_ref,
                bf_ref, q_ref):
    hi = lax.Precision.HIGHEST
    sec = sec_ref[...]
    xwe = jnp.dot(sec, we_ref[...], preferred_element_type=jnp.float32,
                  precision=hi)
    als = jnp.dot(xwe, aes_ref[...], preferred_element_type=jnp.float32,
                  precision=hi)
    ald = jnp.dot(xwe, aed_ref[...], preferred_element_type=jnp.float32,
                  precision=hi)
    iot = lax.broadcasted_iota(jnp.int32, (EI, S), 1)
    srcc = ei_ref[0, :].reshape(EI, 1)
    dstc = ei_ref[1, :].reshape(EI, 1)
    oh_s = (srcc == iot).astype(jnp.float32)
    oh_d = (dstc == iot).astype(jnp.float32)
    e_als = jnp.dot(oh_s, als, preferred_element_type=jnp.float32,
                    precision=hi)
    e_ald = jnp.dot(oh_d, ald, preferred_element_type=jnp.float32,
                    precision=hi)
    t = e_als + e_ald
    e = jnp.where(t >= 0.0, t, 0.2 * t)
    m = jnp.max(jnp.where(oh_d > 0.0, e, -jnp.inf), axis=0, keepdims=True)
    m = jnp.where(jnp.isfinite(m), m, 0.0)
    md = jnp.dot(oh_d, m.reshape(S, 1), preferred_element_type=jnp.float32,
                 precision=hi)
    ex = jnp.exp(e - md)
    den = lax.dot_general(oh_d, ex, (((0,), (0,)), ((), ())),
                          preferred_element_type=jnp.float32,
                          precision=hi) + 1e-16
    dd = jnp.dot(oh_d, den, preferred_element_type=jnp.float32, precision=hi)
    alpha = ex / dd
    xs = jnp.dot(oh_s, xwe, preferred_element_type=jnp.float32, precision=hi)
    msg = alpha * xs
    inter = lax.dot_general(oh_d, msg, (((0,), (0,)), ((), ())),
                            preferred_element_type=jnp.float32,
                            precision=hi) + be_ref[...]
    q_ref[...] = jnp.dot(inter, wf3_ref[...],
                         preferred_element_type=jnp.float32,
                         precision=hi) + bf_ref[...]


def _run_inter(sec, we, aes, aed, be, ei, wf3, bf):
    return pl.pallas_call(
        _inter_body,
        out_shape=jax.ShapeDtypeStruct((S, 1), jnp.float32),
    )(sec, we, aes, aed, be, ei, wf3, bf)


def _fuse_body(seq_ref, intra_ref, sid_ref, q_ref, wf1_ref, wf2_ref, o_ref):
    hi = lax.Precision.HIGHEST
    iot = lax.broadcasted_iota(jnp.int32, (NBLK, S), 1)
    oh = (sid_ref[...] == iot).astype(jnp.float32)
    g = jnp.dot(oh, q_ref[...], preferred_element_type=jnp.float32,
                precision=hi)
    o_ref[...] = (
        jnp.dot(seq_ref[...], wf1_ref[...], preferred_element_type=jnp.float32,
                precision=hi)
        + jnp.dot(intra_ref[...], wf2_ref[...],
                  preferred_element_type=jnp.float32, precision=hi)
        + g)


def _run_fuse(seq, intra, sids, q, wf1, wf2):
    return pl.pallas_call(
        _fuse_body,
        grid=(NGRID,),
        in_specs=[
            pl.BlockSpec((NBLK, H), lambda i: (i, 0)),
            pl.BlockSpec((NBLK, H), lambda i: (i, 0)),
            pl.BlockSpec((NBLK, 1), lambda i: (i, 0)),
            pl.BlockSpec((S, 1), lambda i: (0, 0)),
            pl.BlockSpec((H, 1), lambda i: (0, 0)),
            pl.BlockSpec((H, 1), lambda i: (0, 0)),
        ],
        out_specs=pl.BlockSpec((NBLK, 1), lambda i: (i, 0)),
        out_shape=jax.ShapeDtypeStruct((N, 1), jnp.float32),
    )(seq, intra, sids, q, wf1, wf2)


@jax.jit
def kernel(x, W_ih, W_hh, b_ih, b_hh, Wi, ai_src, ai_dst, bi, We, ae_src,
           ae_dst, be, Wf, bf, intra_edge_index, inter_edge_index,
           sector_ids):
    xt = jnp.swapaxes(x, 0, 1)
    seq, xw, a_s, a_d, mx = _run_gru(
        xt, W_ih, W_hh, b_ih.reshape(1, -1), b_hh.reshape(1, -1), Wi,
        ai_src.reshape(H, 1), ai_dst.reshape(H, 1))
    mx16 = jnp.broadcast_to(mx.reshape(1), (16,))
    acc, den = _run_edges(intra_edge_index.reshape(2, NS, NB, KE),
                          a_s.ravel(), a_d.ravel(), mx16,
                          xw.reshape(2 * N, HC))
    intra, sec = _run_norm(acc, den.reshape(NS, NGRID, NBLK),
                           bi.reshape(1, H), sector_ids.reshape(N, 1))
    q = _run_inter(sec, We, ae_src.reshape(H, 1), ae_dst.reshape(H, 1),
                   be.reshape(1, H), inter_edge_index, Wf[2 * H:],
                   bf.reshape(1, 1))
    out = _run_fuse(seq, intra, sector_ids.reshape(N, 1), q,
                    Wf[:H], Wf[H:2 * H])
    return out.ravel()
